# baseline (speedup 1.0000x reference)
import functools

import numpy as np
import jax
import jax.numpy as jnp
from jax import lax
from jax.experimental import pallas as pl
from jax.experimental.pallas import tpu as pltpu
from jax.experimental.pallas import tpu_sc as plsc

_L = 16
_F = 2
_T = 2 ** 19
_BASE_RES = 16
_SCALE = 1.3819
_D_IN = 3
_OUT_DIM = 64
_N = 262144

_RES = [int(np.floor(_BASE_RES * _SCALE ** l)) for l in range(_L)]
_P1 = int(np.uint32(2654435761).view(np.int32))
_P2 = 805459861
_MASK = _T - 1

_NC = 2
_NS = 16
_NW = _NC * _NS
_PW = _N // _NW
_C = 1024
_NCHUNK = _PW // _C

_FEAT = _L * _F


def _sc_body(x0, x1, x2, tabs, out, xb, idxb, wb, rowsb, outb, sem):
    cid = lax.axis_index("c")
    sid = lax.axis_index("s")
    wid = sid * _NC + cid

    iota = lax.iota(jnp.int32, 16)
    dup = lax.shift_right_logical(iota, 1)
    bit = lax.bitwise_and(iota, 1)

    @pl.loop(0, _NCHUNK)
    def _chunk(ci):
        base = wid * _PW + ci * _C
        for d, xin in enumerate((x0, x1, x2)):
            pltpu.sync_copy(xin.at[pl.ds(base, _C)], xb[d])

        @pl.loop(0, _C // 16)
        def _u(g):
            o = g * 16
            for d in range(3):
                v = xb[d][pl.ds(o, 16)]
                xb[d][pl.ds(o, 16)] = (v + 1.0) * 0.5

        for l in range(_L):
            res = _RES[l]
            resf = np.float32(res)
            lofs = l * _T

            @pl.loop(0, _C // 16)
            def _iw(g):
                o = g * 16
                u0 = xb[0][pl.ds(o, 16)]
                u1 = xb[1][pl.ds(o, 16)]
                u2 = xb[2][pl.ds(o, 16)]
                p0 = u0 * resf
                p1 = u1 * resf
                p2 = u2 * resf
                b0 = p0.astype(jnp.int32)
                b1 = p1.astype(jnp.int32)
                b2 = p2.astype(jnp.int32)
                f0 = p0 - b0.astype(jnp.float32)
                f1 = p1 - b1.astype(jnp.float32)
                f2 = p2 - b2.astype(jnp.float32)
                hx = (b0, jnp.minimum(b0 + 1, res))
                hy = (b1 * _P1, jnp.minimum(b1 + 1, res) * _P1)
                hz = (b2 * _P2, jnp.minimum(b2 + 1, res) * _P2)
                wx = (1.0 - f0, f0)
                wy = (1.0 - f1, f1)
                wz = (1.0 - f2, f2)
                for ox in range(2):
                    for oy in range(2):
                        hxy = lax.bitwise_xor(hx[ox], hy[oy])
                        wxy = wx[ox] * wy[oy]
                        for oz in range(2):
                            c8 = ox * 4 + oy * 2 + oz
                            h = lax.bitwise_and(
                                lax.bitwise_xor(hxy, hz[oz]), _MASK)
                            idxb[c8][pl.ds(o, 16)] = lax.bitwise_or(h, lofs)
                            wb[c8][pl.ds(o, 16)] = wxy * wz[oz]

            cps = [
                pltpu.async_copy(tabs.at[idxb[c8]], rowsb[c8], sem)
                for c8 in range(8)
            ]
            for cp in cps:
                cp.wait()

            colv = bit + 2 * l

            @pl.loop(0, _C // 8)
            def _acc(g2):
                rid = dup + g2 * 8
                for c8 in range(8):
                    wv = plsc.load_gather(wb[c8], [rid])
                    ev = plsc.load_gather(rowsb[c8], [rid, bit])
                    val = wv * ev
                    if c8 == 0:
                        plsc.store_scatter(outb, [rid, colv], val)
                    else:
                        plsc.addupdate_scatter(outb, [rid, colv], val)

        pltpu.sync_copy(outb, out.at[pl.ds(base, _C)])


@functools.lru_cache(maxsize=None)
def _make_sc_call():
    return pl.kernel(
        _sc_body,
        out_type=jax.ShapeDtypeStruct((_N, _FEAT), jnp.float32),
        mesh=plsc.VectorSubcoreMesh(
            core_axis_name="c", subcore_axis_name="s",
            num_cores=_NC, num_subcores=_NS),
        compiler_params=pltpu.CompilerParams(
            needs_layout_passes=False, use_tc_tiling_on_sc=False),
        scratch_types=[
            [pltpu.VMEM((_C,), jnp.float32) for _ in range(3)],
            [pltpu.VMEM((_C,), jnp.int32) for _ in range(8)],
            [pltpu.VMEM((_C,), jnp.float32) for _ in range(8)],
            [pltpu.VMEM((_C, 2), jnp.float32) for _ in range(8)],
            pltpu.VMEM((_C, _FEAT), jnp.float32),
            pltpu.SemaphoreType.DMA,
        ],
    )


_B = 2048


def _tc_body(g_ref, x48_ref, w_ref, asin_ref, acos_ref, b_ref, o_ref):
    lvl = lax.broadcasted_iota(jnp.int32, (1, 3 * _L), 1) // 3
    freq = (1 << lvl).astype(jnp.float32) * np.float32(np.pi)
    args = x48_ref[...] * freq
    s = jnp.sin(args)
    c = jnp.cos(args)
    acc = jnp.dot(g_ref[...], w_ref[...],
                  preferred_element_type=jnp.float32,
                  precision=lax.Precision.HIGHEST)
    acc += jnp.dot(s, asin_ref[...],
                   preferred_element_type=jnp.float32,
                   precision=lax.Precision.HIGHEST)
    acc += jnp.dot(c, acos_ref[...],
                   preferred_element_type=jnp.float32,
                   precision=lax.Precision.HIGHEST)
    o_ref[...] = acc + b_ref[...]


_tc_call = pl.pallas_call(
    _tc_body,
    out_shape=jax.ShapeDtypeStruct((_N, _OUT_DIM), jnp.float32),
    grid=(_N // _B,),
    in_specs=[
        pl.BlockSpec((_B, _FEAT), lambda i: (i, 0)),
        pl.BlockSpec((_B, 3 * _L), lambda i: (i, 0)),
        pl.BlockSpec((_FEAT, _OUT_DIM), lambda i: (0, 0)),
        pl.BlockSpec((3 * _L, _OUT_DIM), lambda i: (0, 0)),
        pl.BlockSpec((3 * _L, _OUT_DIM), lambda i: (0, 0)),
        pl.BlockSpec((1, _OUT_DIM), lambda i: (0, 0)),
    ],
    out_specs=pl.BlockSpec((_B, _OUT_DIM), lambda i: (i, 0)),
)


def kernel(x, tables, ff_proj, W_out, b_out):
    xt = x.T
    grid = _make_sc_call()(xt[0], xt[1], xt[2],
                           tables.reshape(_L * _T, _F))
    x48 = jnp.tile(x, (1, _L))
    wp = W_out.reshape(_L, _F, _OUT_DIM)
    asin = jnp.einsum("ldj,ljo->ldo", ff_proj[:, :3, :], wp)
    acos = jnp.einsum("ldj,ljo->ldo", ff_proj[:, 3:, :], wp)
    return _tc_call(grid, x48, W_out,
                    asin.reshape(3 * _L, _OUT_DIM),
                    acos.reshape(3 * _L, _OUT_DIM),
                    b_out[None, :])


# speedup vs baseline: 6.7985x; 6.7985x over previous
import functools

import numpy as np
import jax
import jax.numpy as jnp
from jax import lax
from jax.experimental import pallas as pl
from jax.experimental.pallas import tpu as pltpu
from jax.experimental.pallas import tpu_sc as plsc

_L = 16
_F = 2
_T = 2 ** 19
_BASE_RES = 16
_SCALE = 1.3819
_D_IN = 3
_OUT_DIM = 64
_N = 262144

_RES = [int(np.floor(_BASE_RES * _SCALE ** l)) for l in range(_L)]
_P1 = int(np.uint32(2654435761).view(np.int32))
_P2 = 805459861
_MASK = _T - 1

_NC = 2
_NS = 16
_PT = _N // _NS
_C = 512
_NCHUNK = _PT // _C
_LH = _L // _NC

_WSROWS = _L * _T * _F // 8
_CVT = _L * _T * _F // _NC // _NS
_SPAN = 4096

_FEAT = _L * _F


def _sc_body(x, tabs, out, ws, xc, xb, idxb, lowb, wb, rowsb, outb,
             cin, cout, sem):
    cid = lax.axis_index("c")
    sid = lax.axis_index("s")

    iota = lax.iota(jnp.int32, 16)
    dup = lax.shift_right_logical(iota, 1)
    bit = lax.bitwise_and(iota, 1)
    oct8 = lax.shift_right_logical(iota, 3)
    low3 = lax.bitwise_and(iota, 7)
    quad = lax.shift_right_logical(iota, 2)
    dupbit = dup + lax.shift_left(bit, 7)

    cstart = cid * (_L * _T * _F // _NC) + sid * _CVT

    @pl.loop(0, _CVT // _SPAN)
    def _span(sp):
        off = cstart + sp * _SPAN
        pltpu.sync_copy(tabs.at[pl.ds(off, _SPAN)], cin)

        @pl.loop(0, _SPAN // 16)
        def _perm(g):
            s = lax.bitwise_or(
                lax.shift_left(lax.shift_right_logical(g, 4), 8),
                lax.shift_left(lax.bitwise_and(g, 15), 3))
            v = plsc.load_gather(cin, [s + dupbit])
            o0 = g * 16
            rv = oct8 + lax.shift_right_logical(o0, 3)
            plsc.store_scatter(cout, [rv, low3], v)

        pltpu.sync_copy(cout, ws.at[pl.ds(lax.shift_right_logical(off, 3),
                                          _SPAN // 8)])

    plsc.subcore_barrier()

    def compute_iw(l, s):
        res = _RES[l]
        resf = np.float32(res)

        @pl.loop(0, _C // 16)
        def _iw(g):
            o = g * 16
            u0 = xb[0][pl.ds(o, 16)]
            u1 = xb[1][pl.ds(o, 16)]
            u2 = xb[2][pl.ds(o, 16)]
            p0 = u0 * resf
            p1 = u1 * resf
            p2 = u2 * resf
            b0 = p0.astype(jnp.int32)
            b1 = p1.astype(jnp.int32)
            b2 = p2.astype(jnp.int32)
            f0 = p0 - b0.astype(jnp.float32)
            f1 = p1 - b1.astype(jnp.float32)
            f2 = p2 - b2.astype(jnp.float32)
            hx = (b0, jnp.minimum(b0 + 1, res))
            hy = (b1 * _P1, jnp.minimum(b1 + 1, res) * _P1)
            hz = (b2 * _P2, jnp.minimum(b2 + 1, res) * _P2)
            wx = (1.0 - f0, f0)
            wy = (1.0 - f1, f1)
            wz = (1.0 - f2, f2)
            for ox in range(2):
                for oy in range(2):
                    hxy = lax.bitwise_xor(hx[ox], hy[oy])
                    wxy = wx[ox] * wy[oy]
                    for oz in range(2):
                        c8 = ox * 4 + oy * 2 + oz
                        h = lax.bitwise_and(
                            lax.bitwise_xor(hxy, hz[oz]), _MASK)
                        row = lax.bitwise_or(
                            lax.shift_right_logical(h, 2), l << 17)
                        idxb[s][pl.ds(c8 * _C + o, 16)] = row
                        lowb[s][pl.ds(c8 * _C + o, 16)] = lax.shift_left(
                            lax.bitwise_and(h, 3), 1)
                        wb[s][pl.ds(c8 * _C + o, 16)] = wxy * wz[oz]

    def gather(s):
        return pltpu.async_copy(ws.at[idxb[s]], rowsb[s], sem)

    def accumulate(l, lh, s):
        colv0 = lax.shift_left(lax.bitwise_and(iota, 3), 4) + 2 * lh
        colv1 = colv0 + 1

        @pl.loop(0, _C // 16)
        def _acc(g2):
            o = g2 * 16
            acc0 = acc1 = None
            for c8 in range(8):
                wv = wb[s][pl.ds(c8 * _C + o, 16)]
                rid = iota + (c8 * _C + o)
                c0 = lowb[s][pl.ds(c8 * _C + o, 16)]
                v0 = plsc.load_gather(rowsb[s], [rid, c0])
                v1 = plsc.load_gather(rowsb[s], [rid, c0 + 1])
                if c8 == 0:
                    acc0, acc1 = wv * v0, wv * v1
                else:
                    acc0, acc1 = acc0 + wv * v0, acc1 + wv * v1
            rowv = quad + (o >> 2)
            plsc.store_scatter(outb, [rowv, colv0], acc0)
            plsc.store_scatter(outb, [rowv, colv1], acc1)

    @pl.loop(0, _NCHUNK)
    def _chunk(ci):
        base = sid * _PT + ci * _C
        pltpu.sync_copy(x.at[pl.ds(base, _C)], xc)

        @pl.loop(0, _C // 16)
        def _u(g):
            o = g * 16
            rid = o + iota
            for d in range(3):
                v = plsc.load_gather(xc, [rid, jnp.full_like(iota, d)])
                xb[d][pl.ds(o, 16)] = (v + 1.0) * 0.5

        for half in range(_NC):
            @pl.when(cid == half)
            def _levels():
                l0 = half * _LH
                compute_iw(l0, 0)
                cps = {0: gather(0)}
                for lh in range(_LH):
                    if lh + 1 < _LH:
                        s_next = (lh + 1) % 2
                        compute_iw(l0 + lh + 1, s_next)
                        cps[lh + 1] = gather(s_next)
                    cps.pop(lh).wait()
                    accumulate(l0 + lh, lh, lh % 2)

        pltpu.sync_copy(
            outb,
            out.at[pl.ds(base // 4, _C // 4), pl.ds(cid * 64, 64)])


@functools.lru_cache(maxsize=None)
def _make_sc_call():
    return pl.kernel(
        _sc_body,
        out_type=(
            jax.ShapeDtypeStruct((_N // 4, 128), jnp.float32),
            jax.ShapeDtypeStruct((_WSROWS, 8), jnp.float32),
        ),
        mesh=plsc.VectorSubcoreMesh(
            core_axis_name="c", subcore_axis_name="s",
            num_cores=_NC, num_subcores=_NS),
        compiler_params=pltpu.CompilerParams(
            needs_layout_passes=False, use_tc_tiling_on_sc=False),
        scratch_types=[
            pltpu.VMEM((_C, 3), jnp.float32),
            [pltpu.VMEM((_C,), jnp.float32) for _ in range(3)],
            [pltpu.VMEM((8 * _C,), jnp.int32) for _ in range(2)],
            [pltpu.VMEM((8 * _C,), jnp.int32) for _ in range(2)],
            [pltpu.VMEM((8 * _C,), jnp.float32) for _ in range(2)],
            [pltpu.VMEM((8 * _C, 8), jnp.float32) for _ in range(2)],
            pltpu.VMEM((_C // 4, 64), jnp.float32),
            pltpu.VMEM((_SPAN,), jnp.float32),
            pltpu.VMEM((_SPAN // 8, 8), jnp.float32),
            pltpu.SemaphoreType.DMA,
        ],
    )


_B4 = 512


def _tc_body(g_ref, x_ref, w_ref, asin_ref, acos_ref, b_ref, o_ref):
    lvl = lax.broadcasted_iota(jnp.int32, (1, 12 * _L), 1) // 12
    freq = (1 << lvl).astype(jnp.float32) * np.float32(np.pi)
    args = x_ref[...] * freq
    s = jnp.sin(args)
    c = jnp.cos(args)
    acc = jnp.dot(g_ref[...], w_ref[...],
                  preferred_element_type=jnp.float32,
                  precision=lax.Precision.HIGHEST)
    acc += jnp.dot(s, asin_ref[...],
                   preferred_element_type=jnp.float32,
                   precision=lax.Precision.HIGHEST)
    acc += jnp.dot(c, acos_ref[...],
                   preferred_element_type=jnp.float32,
                   precision=lax.Precision.HIGHEST)
    o_ref[...] = acc + b_ref[...]


_tc_call = pl.pallas_call(
    _tc_body,
    out_shape=jax.ShapeDtypeStruct((_N // 4, 4 * _OUT_DIM), jnp.float32),
    grid=(_N // 4 // _B4,),
    in_specs=[
        pl.BlockSpec((_B4, 128), lambda i: (i, 0)),
        pl.BlockSpec((_B4, 12 * _L), lambda i: (i, 0)),
        pl.BlockSpec((128, 4 * _OUT_DIM), lambda i: (0, 0)),
        pl.BlockSpec((12 * _L, 4 * _OUT_DIM), lambda i: (0, 0)),
        pl.BlockSpec((12 * _L, 4 * _OUT_DIM), lambda i: (0, 0)),
        pl.BlockSpec((1, 4 * _OUT_DIM), lambda i: (0, 0)),
    ],
    out_specs=pl.BlockSpec((_B4, 4 * _OUT_DIM), lambda i: (i, 0)),
)

_WPERM = np.array([
    32 * ((c >> 4) & 3) + 2 * (8 * (c >> 6) + ((c & 15) >> 1)) + (c & 1)
    for c in range(128)])


def kernel(x, tables, ff_proj, W_out, b_out):
    traw = jnp.transpose(
        tables.reshape(_L, _T // 128, 128, _F), (0, 1, 3, 2)
    ).reshape(_L * _T * _F)
    grid, _ = _make_sc_call()(x, traw)
    x192 = jnp.tile(x.reshape(_N // 4, 12), (1, _L))
    eye4 = jnp.eye(4, dtype=jnp.float32)
    wp = W_out.reshape(_L, _F, _OUT_DIM)
    asin = jnp.einsum("ldj,ljo->ldo", ff_proj[:, :3, :], wp)
    acos = jnp.einsum("ldj,ljo->ldo", ff_proj[:, 3:, :], wp)
    asin4 = (eye4[None, :, None, :, None]
             * asin[:, None, :, None, :]).reshape(12 * _L, 4 * _OUT_DIM)
    acos4 = (eye4[None, :, None, :, None]
             * acos[:, None, :, None, :]).reshape(12 * _L, 4 * _OUT_DIM)
    w4 = jnp.kron(eye4, W_out)[_WPERM, :]
    b4 = jnp.tile(b_out, 4)[None, :]
    out4 = _tc_call(grid, x192, w4, asin4, acos4, b4)
    return out4.reshape(_N, _OUT_DIM)


# speedup vs baseline: 6.8256x; 1.0040x over previous
import functools

import numpy as np
import jax
import jax.numpy as jnp
from jax import lax
from jax.experimental import pallas as pl
from jax.experimental.pallas import tpu as pltpu
from jax.experimental.pallas import tpu_sc as plsc

_L = 16
_F = 2
_T = 2 ** 19
_BASE_RES = 16
_SCALE = 1.3819
_D_IN = 3
_OUT_DIM = 64
_N = 262144

_RES = [int(np.floor(_BASE_RES * _SCALE ** l)) for l in range(_L)]
_P1 = int(np.uint32(2654435761).view(np.int32))
_P2 = 805459861
_MASK = _T - 1

_NC = 2
_NS = 16
_NH = _N // 2
_PT = _NH // _NS
_C = 512
_NCHUNK = _PT // _C
_LH = _L // _NC

_WSROWS = _L * _T * _F // 8
_CVT = _L * _T * _F // _NC // _NS
_SPAN = 4096

_FEAT = _L * _F


def _sc_body(convert, x, tabs, out, ws, xc, xb, idxb, lowb, wb, rowsb,
             outb, cin, cout, sem):
    cid = lax.axis_index("c")
    sid = lax.axis_index("s")

    iota = lax.iota(jnp.int32, 16)
    dup = lax.shift_right_logical(iota, 1)
    bit = lax.bitwise_and(iota, 1)
    oct8 = lax.shift_right_logical(iota, 3)
    low3 = lax.bitwise_and(iota, 7)
    quad = lax.shift_right_logical(iota, 2)
    dupbit = dup + lax.shift_left(bit, 7)

    if convert:
        cstart = cid * (_L * _T * _F // _NC) + sid * _CVT

        @pl.loop(0, _CVT // _SPAN)
        def _span(sp):
            off = cstart + sp * _SPAN
            pltpu.sync_copy(tabs.at[pl.ds(off, _SPAN)], cin)

            @pl.loop(0, _SPAN // 16)
            def _perm(g):
                s = lax.bitwise_or(
                    lax.shift_left(lax.shift_right_logical(g, 4), 8),
                    lax.shift_left(lax.bitwise_and(g, 15), 3))
                v = plsc.load_gather(cin, [s + dupbit])
                o0 = g * 16
                rv = oct8 + lax.shift_right_logical(o0, 3)
                plsc.store_scatter(cout, [rv, low3], v)

            pltpu.sync_copy(
                cout, ws.at[pl.ds(lax.shift_right_logical(off, 3),
                                  _SPAN // 8)])

        plsc.subcore_barrier()

    def compute_iw(l, s):
        res = _RES[l]
        resf = np.float32(res)

        @pl.loop(0, _C // 16)
        def _iw(g):
            o = g * 16
            u0 = xb[0][pl.ds(o, 16)]
            u1 = xb[1][pl.ds(o, 16)]
            u2 = xb[2][pl.ds(o, 16)]
            p0 = u0 * resf
            p1 = u1 * resf
            p2 = u2 * resf
            b0 = p0.astype(jnp.int32)
            b1 = p1.astype(jnp.int32)
            b2 = p2.astype(jnp.int32)
            f0 = p0 - b0.astype(jnp.float32)
            f1 = p1 - b1.astype(jnp.float32)
            f2 = p2 - b2.astype(jnp.float32)
            hx = (b0, jnp.minimum(b0 + 1, res))
            hy = (b1 * _P1, jnp.minimum(b1 + 1, res) * _P1)
            hz = (b2 * _P2, jnp.minimum(b2 + 1, res) * _P2)
            wx = (1.0 - f0, f0)
            wy = (1.0 - f1, f1)
            wz = (1.0 - f2, f2)
            for ox in range(2):
                for oy in range(2):
                    hxy = lax.bitwise_xor(hx[ox], hy[oy])
                    wxy = wx[ox] * wy[oy]
                    for oz in range(2):
                        c8 = ox * 4 + oy * 2 + oz
                        h = lax.bitwise_and(
                            lax.bitwise_xor(hxy, hz[oz]), _MASK)
                        row = lax.bitwise_or(
                            lax.shift_right_logical(h, 2), l << 17)
                        idxb[s][pl.ds(c8 * _C + o, 16)] = row
                        lowb[s][pl.ds(c8 * _C + o, 16)] = lax.shift_left(
                            lax.bitwise_and(h, 3), 1)
                        wb[s][pl.ds(c8 * _C + o, 16)] = wxy * wz[oz]

    def gather(s):
        return pltpu.async_copy(ws.at[idxb[s]], rowsb[s], sem)

    def accumulate(l, lh, s):
        colv0 = lax.shift_left(lax.bitwise_and(iota, 3), 4) + 2 * lh
        colv1 = colv0 + 1

        @pl.loop(0, _C // 16)
        def _acc(g2):
            o = g2 * 16
            acc0 = acc1 = None
            for c8 in range(8):
                wv = wb[s][pl.ds(c8 * _C + o, 16)]
                rid = iota + (c8 * _C + o)
                c0 = lowb[s][pl.ds(c8 * _C + o, 16)]
                v0 = plsc.load_gather(rowsb[s], [rid, c0])
                v1 = plsc.load_gather(rowsb[s], [rid, c0 + 1])
                if c8 == 0:
                    acc0, acc1 = wv * v0, wv * v1
                else:
                    acc0, acc1 = acc0 + wv * v0, acc1 + wv * v1
            rowv = quad + (o >> 2)
            plsc.store_scatter(outb, [rowv, colv0], acc0)
            plsc.store_scatter(outb, [rowv, colv1], acc1)

    @pl.loop(0, _NCHUNK)
    def _chunk(ci):
        base = sid * _PT + ci * _C
        pltpu.sync_copy(x.at[pl.ds(base, _C)], xc)

        @pl.loop(0, _C // 16)
        def _u(g):
            o = g * 16
            rid = o + iota
            for d in range(3):
                v = plsc.load_gather(xc, [rid, jnp.full_like(iota, d)])
                xb[d][pl.ds(o, 16)] = (v + 1.0) * 0.5

        for half in range(_NC):
            @pl.when(cid == half)
            def _levels():
                l0 = half * _LH
                compute_iw(l0, 0)
                cps = {0: gather(0)}
                for lh in range(_LH):
                    if lh + 1 < _LH:
                        s_next = (lh + 1) % 2
                        compute_iw(l0 + lh + 1, s_next)
                        cps[lh + 1] = gather(s_next)
                    cps.pop(lh).wait()
                    accumulate(l0 + lh, lh, lh % 2)

        pltpu.sync_copy(
            outb,
            out.at[pl.ds(base // 4, _C // 4), pl.ds(cid * 64, 64)])


_SC_SCRATCH = [
    pltpu.VMEM((_C, 3), jnp.float32),
    [pltpu.VMEM((_C,), jnp.float32) for _ in range(3)],
    [pltpu.VMEM((8 * _C,), jnp.int32) for _ in range(2)],
    [pltpu.VMEM((8 * _C,), jnp.int32) for _ in range(2)],
    [pltpu.VMEM((8 * _C,), jnp.float32) for _ in range(2)],
    [pltpu.VMEM((8 * _C, 8), jnp.float32) for _ in range(2)],
    pltpu.VMEM((_C // 4, 64), jnp.float32),
    pltpu.VMEM((_SPAN,), jnp.float32),
    pltpu.VMEM((_SPAN // 8, 8), jnp.float32),
    pltpu.SemaphoreType.DMA,
]


@functools.lru_cache(maxsize=None)
def _make_sc_calls():
    mesh = dict(core_axis_name="c", subcore_axis_name="s",
                num_cores=_NC, num_subcores=_NS)
    params = pltpu.CompilerParams(
        needs_layout_passes=False, use_tc_tiling_on_sc=False)
    sc_a = pl.kernel(
        functools.partial(_sc_body, True),
        out_type=(
            jax.ShapeDtypeStruct((_NH // 4, 128), jnp.float32),
            jax.ShapeDtypeStruct((_WSROWS, 8), jnp.float32),
        ),
        mesh=plsc.VectorSubcoreMesh(**mesh),
        compiler_params=params,
        scratch_types=_SC_SCRATCH,
    )

    def body_b(x, ws, out, *scratch):
        _sc_body(False, x, ws, out, ws, *scratch)

    sc_b = pl.kernel(
        body_b,
        out_type=jax.ShapeDtypeStruct((_NH // 4, 128), jnp.float32),
        mesh=plsc.VectorSubcoreMesh(**mesh),
        compiler_params=params,
        scratch_types=_SC_SCRATCH,
    )
    return sc_a, sc_b


_B4 = 512


def _tc_body(g_ref, x_ref, w_ref, asin_ref, acos_ref, b_ref, o_ref):
    lvl = lax.broadcasted_iota(jnp.int32, (1, 12 * _L), 1) // 12
    freq = (1 << lvl).astype(jnp.float32) * np.float32(np.pi)
    args = x_ref[...] * freq
    s = jnp.sin(args)
    c = jnp.cos(args)
    acc = jnp.dot(g_ref[...], w_ref[...],
                  preferred_element_type=jnp.float32,
                  precision=lax.Precision.HIGHEST)
    acc += jnp.dot(s, asin_ref[...],
                   preferred_element_type=jnp.float32,
                   precision=lax.Precision.HIGHEST)
    acc += jnp.dot(c, acos_ref[...],
                   preferred_element_type=jnp.float32,
                   precision=lax.Precision.HIGHEST)
    o_ref[...] = acc + b_ref[...]


_tc_call = pl.pallas_call(
    _tc_body,
    out_shape=jax.ShapeDtypeStruct((_NH // 4, 4 * _OUT_DIM), jnp.float32),
    grid=(_NH // 4 // _B4,),
    in_specs=[
        pl.BlockSpec((_B4, 128), lambda i: (i, 0)),
        pl.BlockSpec((_B4, 12 * _L), lambda i: (i, 0)),
        pl.BlockSpec((128, 4 * _OUT_DIM), lambda i: (0, 0)),
        pl.BlockSpec((12 * _L, 4 * _OUT_DIM), lambda i: (0, 0)),
        pl.BlockSpec((12 * _L, 4 * _OUT_DIM), lambda i: (0, 0)),
        pl.BlockSpec((1, 4 * _OUT_DIM), lambda i: (0, 0)),
    ],
    out_specs=pl.BlockSpec((_B4, 4 * _OUT_DIM), lambda i: (i, 0)),
)

_WPERM = np.array([
    32 * ((c >> 4) & 3) + 2 * (8 * (c >> 6) + ((c & 15) >> 1)) + (c & 1)
    for c in range(128)])


def kernel(x, tables, ff_proj, W_out, b_out):
    traw = jnp.transpose(
        tables.reshape(_L, _T // 128, 128, _F), (0, 1, 3, 2)
    ).reshape(_L * _T * _F)
    sc_a, sc_b = _make_sc_calls()
    grid0, ws = sc_a(x[:_NH], traw)
    grid1 = sc_b(x[_NH:], ws)
    x192 = jnp.tile(x.reshape(_N // 4, 12), (1, _L))
    eye4 = jnp.eye(4, dtype=jnp.float32)
    wp = W_out.reshape(_L, _F, _OUT_DIM)
    asin = jnp.einsum("ldj,ljo->ldo", ff_proj[:, :3, :], wp)
    acos = jnp.einsum("ldj,ljo->ldo", ff_proj[:, 3:, :], wp)
    asin4 = (eye4[None, :, None, :, None]
             * asin[:, None, :, None, :]).reshape(12 * _L, 4 * _OUT_DIM)
    acos4 = (eye4[None, :, None, :, None]
             * acos[:, None, :, None, :]).reshape(12 * _L, 4 * _OUT_DIM)
    w4 = jnp.kron(eye4, W_out)[_WPERM, :]
    b4 = jnp.tile(b_out, 4)[None, :]
    out0 = _tc_call(grid0, x192[:_NH // 4], w4, asin4, acos4, b4)
    out1 = _tc_call(grid1, x192[_NH // 4:], w4, asin4, acos4, b4)
    return jnp.concatenate([out0, out1], axis=0).reshape(_N, _OUT_DIM)
